# baseline (speedup 1.0000x reference)
import jax
import jax.numpy as jnp
from jax.experimental import pallas as pl
from jax.experimental.pallas import tpu as pltpu

_GRID = 2


def _copy_kernel(in_ref, out_ref):
    out_ref[...] = in_ref[...]


def kernel(inputs_embeds, embed_table):
    del embed_table
    b, s, h = inputs_embeds.shape
    n = b * s * h
    x = inputs_embeds.reshape(n)
    chunk = n // _GRID
    out = pl.pallas_call(
        _copy_kernel,
        grid=(_GRID,),
        in_specs=[pl.BlockSpec((chunk,), lambda i: (i,))],
        out_specs=pl.BlockSpec((chunk,), lambda i: (i,)),
        out_shape=jax.ShapeDtypeStruct((n,), inputs_embeds.dtype),
    )(x)
    return out.reshape(b, s, h)


# speedup vs baseline: 1.5888x; 1.5888x over previous
import jax
import jax.numpy as jnp
from jax.experimental import pallas as pl
from jax.experimental.pallas import tpu as pltpu

_GRID = 2


def _copy_kernel(in_ref, out_ref):
    out_ref[...] = in_ref[...]


def kernel(inputs_embeds, embed_table):
    del embed_table
    b, s, h = inputs_embeds.shape
    nb = b // _GRID
    return pl.pallas_call(
        _copy_kernel,
        grid=(_GRID,),
        in_specs=[pl.BlockSpec((nb, s, h), lambda i: (i, 0, 0))],
        out_specs=pl.BlockSpec((nb, s, h), lambda i: (i, 0, 0)),
        out_shape=jax.ShapeDtypeStruct((b, s, h), inputs_embeds.dtype),
        compiler_params=pltpu.CompilerParams(
            needs_layout_passes=False,
        ),
    )(inputs_embeds)


# speedup vs baseline: 10.6688x; 6.7148x over previous
import jax
import jax.numpy as jnp
from jax.experimental import pallas as pl
from jax.experimental.pallas import tpu as pltpu

_GRID = 2


def _copy_kernel(in_ref, out_ref):
    out_ref[...] = in_ref[...]


def kernel(inputs_embeds, embed_table):
    del embed_table
    b, s, h = inputs_embeds.shape
    x = inputs_embeds.transpose(0, 2, 1)
    nb = b // _GRID
    out = pl.pallas_call(
        _copy_kernel,
        grid=(_GRID,),
        in_specs=[pl.BlockSpec((nb, h, s), lambda i: (i, 0, 0))],
        out_specs=pl.BlockSpec((nb, h, s), lambda i: (i, 0, 0)),
        out_shape=jax.ShapeDtypeStruct((b, h, s), inputs_embeds.dtype),
    )(x)
    return out.transpose(0, 2, 1)
